# baseline (speedup 1.0000x reference)
import functools

import jax
import jax.numpy as jnp
from jax import lax
from jax.experimental import pallas as pl
from jax.experimental.pallas import tpu as pltpu
from jax.experimental.pallas import tpu_sc as plsc

N_NODES = 50000
N_EDGES = 800000
D = 64
H = 32

NS = 16
CHUNK = 128
PHASES = 12
EDGES_TILE = N_EDGES // NS
NG = EDGES_TILE // CHUNK
NG_PIPE = (NG // PHASES) * PHASES
TAIL = EDGES_TILE - NG * CHUNK
E4_HALF = N_EDGES // 4

ROWS_PER_TILE = (N_NODES // (8 * NS)) * 8
ROWS_REM = N_NODES - NS * ROWS_PER_TILE
ZR = 240
ACC_ROWS = N_NODES


def _node_proj(x, wp, bp, wm):
    n, d_in = x.shape
    bn = 5000
    nb = n // bn

    def body(x_ref, wp_ref, bp_ref, wm_ref, hv_ref, pcat_ref):
        i = pl.program_id(0)
        hv = jnp.maximum(
            jnp.dot(x_ref[...], wp_ref[...], preferred_element_type=jnp.float32)
            + bp_ref[...], 0.0)
        hv_ref[...] = hv
        p = jnp.dot(hv, wm_ref[...], preferred_element_type=jnp.float32)

        @pl.when(i < nb)
        def _():
            pcat_ref[...] = p[:, :H]

        @pl.when(i >= nb)
        def _():
            pcat_ref[...] = p[:, H:]

    return pl.pallas_call(
        body,
        grid=(2 * nb,),
        in_specs=[
            pl.BlockSpec((bn, d_in), lambda i: (i % nb, 0)),
            pl.BlockSpec((d_in, D), lambda i: (0, 0)),
            pl.BlockSpec((1, D), lambda i: (0, 0)),
            pl.BlockSpec((D, D), lambda i: (0, 0)),
        ],
        out_specs=[
            pl.BlockSpec((bn, D), lambda i: (i % nb, 0)),
            pl.BlockSpec((bn, H), lambda i: (i, 0)),
        ],
        out_shape=[
            jax.ShapeDtypeStruct((n, D), jnp.float32),
            jax.ShapeDtypeStruct((2 * n, H), jnp.float32),
        ],
    )(x, wp, bp, wm)


def _edge_tbl(ef4, wlo, whi, blo, bhi):
    bn = 2000
    nb = E4_HALF // bn

    def body(ef_ref, wlo_ref, whi_ref, blo_ref, bhi_ref, out_ref):
        j = pl.program_id(0)
        x = ef_ref[...]

        @pl.when(j < nb)
        def _():
            out_ref[...] = jnp.dot(
                x, wlo_ref[...], preferred_element_type=jnp.float32
            ) + blo_ref[...]

        @pl.when(j >= nb)
        def _():
            out_ref[...] = jnp.dot(
                x, whi_ref[...], preferred_element_type=jnp.float32
            ) + bhi_ref[...]

    return pl.pallas_call(
        body,
        grid=(2 * nb,),
        in_specs=[
            pl.BlockSpec((bn, 48), lambda j: (j % nb, 0)),
            pl.BlockSpec((48, 128), lambda j: (0, 0)),
            pl.BlockSpec((48, 128), lambda j: (0, 0)),
            pl.BlockSpec((1, 128), lambda j: (0, 0)),
            pl.BlockSpec((1, 128), lambda j: (0, 0)),
        ],
        out_specs=pl.BlockSpec((bn, 128), lambda j: (j, 0)),
        out_shape=jax.ShapeDtypeStruct((2 * E4_HALF, 128), jnp.float32),
    )(ef4, wlo, whi, blo, bhi)


def _update(hv, slo, shi, w1, w2a, w2b, b, wp):
    n = hv.shape[0]
    bn = 5000
    nb = n // bn

    def body(hv_ref, slo_ref, shi_ref, w1_ref, w2a_ref, w2b_ref, b_ref, wp_ref,
             o_ref, pcat_ref):
        i = pl.program_id(0)
        acc = jnp.dot(hv_ref[...], w1_ref[...], preferred_element_type=jnp.float32)
        acc += jnp.dot(slo_ref[...], w2a_ref[...], preferred_element_type=jnp.float32)
        acc += jnp.dot(shi_ref[...], w2b_ref[...], preferred_element_type=jnp.float32)
        hv_new = jnp.maximum(acc + b_ref[...], 0.0)
        o_ref[...] = hv_new
        p = jnp.dot(hv_new, wp_ref[...], preferred_element_type=jnp.float32)

        @pl.when(i < nb)
        def _():
            pcat_ref[...] = p[:, :H]

        @pl.when(i >= nb)
        def _():
            pcat_ref[...] = p[:, H:]

    return pl.pallas_call(
        body,
        grid=(2 * nb,),
        in_specs=[
            pl.BlockSpec((bn, D), lambda i: (i % nb, 0)),
            pl.BlockSpec((bn, H), lambda i: (i % nb, 0)),
            pl.BlockSpec((bn, H), lambda i: (i % nb, 0)),
            pl.BlockSpec((D, D), lambda i: (0, 0)),
            pl.BlockSpec((H, D), lambda i: (0, 0)),
            pl.BlockSpec((H, D), lambda i: (0, 0)),
            pl.BlockSpec((1, D), lambda i: (0, 0)),
            pl.BlockSpec((D, D), lambda i: (0, 0)),
        ],
        out_specs=[
            pl.BlockSpec((bn, D), lambda i: (i % nb, 0)),
            pl.BlockSpec((bn, H), lambda i: (i, 0)),
        ],
        out_shape=[
            jax.ShapeDtypeStruct((n, D), jnp.float32),
            jax.ShapeDtypeStruct((2 * n, H), jnp.float32),
        ],
    )(hv, slo, shi, w1, w2a, w2b, b, wp)


def _final(hv, hlo, hhi, ws):
    n = hv.shape[0]
    bn = 5000

    def body(hv_ref, hlo_ref, hhi_ref, ws_ref, o_ref):
        s = jnp.dot(hv_ref[...], ws_ref[...], preferred_element_type=jnp.float32)
        o_ref[...] = jnp.concatenate(
            [hlo_ref[...] * s[:, :H], hhi_ref[...] * s[:, H:]], axis=1)

    return pl.pallas_call(
        body,
        grid=(n // bn,),
        in_specs=[
            pl.BlockSpec((bn, D), lambda i: (i, 0)),
            pl.BlockSpec((bn, H), lambda i: (i, 0)),
            pl.BlockSpec((bn, H), lambda i: (i, 0)),
            pl.BlockSpec((D, D), lambda i: (0, 0)),
        ],
        out_specs=pl.BlockSpec((bn, D), lambda i: (i, 0)),
        out_shape=jax.ShapeDtypeStruct((n, D), jnp.float32),
    )(hv, hlo, hhi, ws)


def _make_sc_segment(mul: bool):
    mesh = plsc.VectorSubcoreMesh(core_axis_name="c", subcore_axis_name="s",
                                  num_cores=2, num_subcores=NS)

    @functools.partial(
        pl.kernel,
        out_type=[
            jax.ShapeDtypeStruct((N_NODES, H), jnp.float32),
            jax.ShapeDtypeStruct((N_NODES, H), jnp.float32),
        ],
        mesh=mesh,
        compiler_params=pltpu.CompilerParams(use_tc_tiling_on_sc=False),
        scratch_types=[
            pltpu.VMEM((2, CHUNK), jnp.int32),
            pltpu.VMEM((4, CHUNK), jnp.int32),
            pltpu.VMEM((TAIL,), jnp.int32),
            pltpu.VMEM((TAIL,), jnp.int32),
            pltpu.VMEM((3, CHUNK, H), jnp.float32),
            pltpu.VMEM((2, CHUNK // 4, 128), jnp.float32),
            pltpu.VMEM((ZR, H), jnp.float32),
            pltpu.VMEM_SHARED((ACC_ROWS, H), jnp.float32),
            [pltpu.SemaphoreType.DMA] * 3,
            [pltpu.SemaphoreType.DMA] * 2,
            [pltpu.SemaphoreType.DMA] * 3,
            [pltpu.SemaphoreType.DMA] * 2,
            [pltpu.SemaphoreType.DMA] * 4,
        ],
    )
    def seg(src_ref, dst_ref, tcat, ecat, out_lo, out_hi,
            sidx, didx, tsidx, tdidx, prow, erow, zrow, acc,
            gsems, esems, ssems, sisems, disems):
        c = lax.axis_index("c")
        s = lax.axis_index("s")

        zeros16 = jnp.zeros((16,), jnp.float32)

        @pl.loop(0, ZR)
        def _zfill(i):
            zrow[i, pl.ds(0, 16)] = zeros16
            zrow[i, pl.ds(16, 16)] = zeros16

        def zcopy(dst):
            return pltpu.make_async_copy(zrow, dst, gsems[0])

        for j in range(ROWS_PER_TILE // ZR):
            zcopy(acc.at[pl.ds(s * ROWS_PER_TILE + j * ZR, ZR)]).start()

        @pl.when(s == 0)
        def _ztail():
            pltpu.async_copy(zrow.at[pl.ds(0, ROWS_REM)],
                             acc.at[pl.ds(NS * ROWS_PER_TILE, ROWS_REM)],
                             gsems[0])

        for j in range(ROWS_PER_TILE // ZR):
            zcopy(acc.at[pl.ds(s * ROWS_PER_TILE + j * ZR, ZR)]).wait()

        @pl.when(s == 0)
        def _ztail2():
            pltpu.make_async_copy(
                zrow.at[pl.ds(0, ROWS_REM)],
                acc.at[pl.ds(NS * ROWS_PER_TILE, ROWS_REM)], gsems[0]).wait()

        off16 = jnp.full((16,), c * N_NODES, dtype=jnp.int32)
        ebase = s * EDGES_TILE
        ebase4 = c * E4_HALF + s * (EDGES_TILE // 4)

        def isrc(g, slot):
            return pltpu.make_async_copy(
                src_ref.at[pl.ds(ebase + g * CHUNK, CHUNK)], sidx.at[slot],
                sisems[slot])

        def idst(g, slot):
            return pltpu.make_async_copy(
                dst_ref.at[pl.ds(ebase + g * CHUNK, CHUNK)], didx.at[slot],
                disems[slot])

        def shift_src(slot):
            for j in range(CHUNK // 16):
                sidx[slot, pl.ds(j * 16, 16)] = (
                    sidx[slot, pl.ds(j * 16, 16)] + off16)

        def gather(sp, si):
            return pltpu.make_async_copy(
                tcat.at[sidx.at[si]], prow.at[sp], gsems[sp])

        def erows(g, slot):
            return pltpu.make_async_copy(
                ecat.at[pl.ds(ebase4 + g * (CHUNK // 4), CHUNK // 4)],
                erow.at[slot], esems[slot])

        def scat_start(sp, slot4):
            pltpu.async_copy(prow.at[sp], acc.at[didx.at[slot4]],
                             ssems[sp], add=True)

        def scat_wait(sp, slot4):
            pltpu.make_async_copy(
                prow.at[sp], acc.at[didx.at[slot4]], ssems[sp]).wait()

        def combine(sp, se):
            @plsc.parallel_loop(0, CHUNK // 4, unroll=2)
            def _rows(r):
                for jj in range(4):
                    for off in (0, 16):
                        e = erow[se, r, pl.ds(32 * jj + off, 16)]
                        pv = prow[sp, 4 * r + jj, pl.ds(off, 16)]
                        if mul:
                            prow[sp, 4 * r + jj, pl.ds(off, 16)] = e * pv
                        else:
                            prow[sp, 4 * r + jj, pl.ds(off, 16)] = (
                                jnp.maximum(e + pv, 0.0))

        plsc.subcore_barrier()

        isrc(0, 0).start()
        idst(0, 0).start()
        isrc(1, 1).start()
        idst(1, 1).start()
        isrc(0, 0).wait()
        shift_src(0)
        gather(0, 0).start()
        erows(0, 0).start()

        @pl.loop(0, NG_PIPE // PHASES)
        def _groups(k):
            for p in range(PHASES):
                s2 = p % 2
                s3 = p % 3
                s4 = p % 4
                g = k * PHASES + p

                gather(s3, s2).wait()
                erows(g, s2).wait()
                idst(g, s4).wait()

                @pl.when(g >= 2)
                def _():
                    scat_wait((p + 1) % 3, (p + 2) % 4)

                @pl.when(g + 2 < NG_PIPE)
                def _():
                    isrc(g + 2, s2).start()
                    idst(g + 2, (p + 2) % 4).start()

                @pl.when(g + 1 < NG_PIPE)
                def _():
                    isrc(g + 1, (p + 1) % 2).wait()
                    shift_src((p + 1) % 2)
                    gather((p + 1) % 3, (p + 1) % 2).start()
                    erows(g + 1, (p + 1) % 2).start()

                combine(s3, s2)
                scat_start(s3, s4)

        scat_wait((NG_PIPE - 2) % 3, (NG_PIPE - 2) % 4)
        scat_wait((NG_PIPE - 1) % 3, (NG_PIPE - 1) % 4)

        @pl.loop(NG_PIPE, NG)
        def _tailg(g):
            pltpu.sync_copy(src_ref.at[pl.ds(ebase + g * CHUNK, CHUNK)],
                            sidx.at[0])
            pltpu.sync_copy(dst_ref.at[pl.ds(ebase + g * CHUNK, CHUNK)],
                            didx.at[0])
            shift_src(0)
            gather(0, 0).start()
            erows(g, 0).start()
            gather(0, 0).wait()
            erows(g, 0).wait()
            combine(0, 0)
            scat_start(0, 0)
            scat_wait(0, 0)

        toff = ebase + NG * CHUNK
        pltpu.sync_copy(src_ref.at[pl.ds(toff, TAIL)], tsidx)
        pltpu.sync_copy(dst_ref.at[pl.ds(toff, TAIL)], tdidx)
        for j in range(TAIL // 16):
            tsidx[pl.ds(j * 16, 16)] = tsidx[pl.ds(j * 16, 16)] + off16
        tg = pltpu.make_async_copy(
            tcat.at[tsidx], prow.at[0, pl.ds(0, TAIL)], gsems[0])
        te = pltpu.make_async_copy(
            ecat.at[pl.ds(ebase4 + NG * (CHUNK // 4), TAIL // 4)],
            erow.at[0, pl.ds(0, TAIL // 4)], esems[0])
        tg.start()
        te.start()
        tg.wait()
        te.wait()

        @pl.loop(0, TAIL // 4)
        def _trows(r):
            for jj in range(4):
                for off in (0, 16):
                    e = erow[0, r, pl.ds(32 * jj + off, 16)]
                    pv = prow[0, 4 * r + jj, pl.ds(off, 16)]
                    if mul:
                        prow[0, 4 * r + jj, pl.ds(off, 16)] = e * pv
                    else:
                        prow[0, 4 * r + jj, pl.ds(off, 16)] = (
                            jnp.maximum(e + pv, 0.0))

        ts = pltpu.make_async_copy(
            prow.at[0, pl.ds(0, TAIL)], acc.at[tdidx], ssems[0])
        pltpu.async_copy(prow.at[0, pl.ds(0, TAIL)], acc.at[tdidx],
                         ssems[0], add=True)
        ts.wait()

        plsc.subcore_barrier()
        r0 = s * ROWS_PER_TILE
        rtail = NS * ROWS_PER_TILE

        @pl.when(c == 0)
        def _():
            pltpu.sync_copy(acc.at[pl.ds(r0, ROWS_PER_TILE)],
                            out_lo.at[pl.ds(r0, ROWS_PER_TILE)])

            @pl.when(s == 0)
            def _():
                pltpu.sync_copy(acc.at[pl.ds(rtail, ROWS_REM)],
                                out_lo.at[pl.ds(rtail, ROWS_REM)])

        @pl.when(c == 1)
        def _():
            pltpu.sync_copy(acc.at[pl.ds(r0, ROWS_PER_TILE)],
                            out_hi.at[pl.ds(r0, ROWS_PER_TILE)])

            @pl.when(s == 0)
            def _():
                pltpu.sync_copy(acc.at[pl.ds(rtail, ROWS_REM)],
                                out_hi.at[pl.ds(rtail, ROWS_REM)])

    return seg


_seg_sum_relu = _make_sc_segment(mul=False)
_seg_sum_mul = _make_sc_segment(mul=True)


def kernel(node_feats, edge_feats, edge_index, W_proj_in, b_proj_in,
           W_msg, b_msg, W_new, b_new, W_node_msg, W_edge_msg, W_self):
    src = edge_index[0].astype(jnp.int32)
    dst = edge_index[1].astype(jnp.int32)
    ef4 = edge_feats.reshape(N_EDGES // 4, 48)

    bd = jax.scipy.linalg.block_diag
    we, wm = W_msg[D:], W_edge_msg
    welo = bd(*([we[:, :H]] * 4))
    wehi = bd(*([we[:, H:]] * 4))
    wmlo = bd(*([wm[:, :H]] * 4))
    wmhi = bd(*([wm[:, H:]] * 4))
    belo = jnp.tile(b_msg[:H], 4).reshape(1, 128)
    behi = jnp.tile(b_msg[H:], 4).reshape(1, 128)

    hv, pcat = _node_proj(node_feats, W_proj_in, b_proj_in.reshape(1, D),
                          W_msg[:D])
    ecat = _edge_tbl(ef4, welo, wehi, belo, behi)
    zb = jnp.zeros((1, 128), jnp.float32)
    mcat = _edge_tbl(ef4, wmlo, wmhi, zb, zb)

    for layer in range(4):
        slo, shi = _seg_sum_relu(src, dst, pcat, ecat)
        wp = W_msg[:D] if layer < 3 else W_node_msg
        hv, pcat = _update(hv, slo, shi, W_new[:D], W_new[D:D + H],
                           W_new[D + H:], b_new.reshape(1, D), wp)

    hlo, hhi = _seg_sum_mul(src, dst, pcat, mcat)
    return _final(hv, hlo, hhi, W_self)


# speedup vs baseline: 1.0055x; 1.0055x over previous
import functools

import jax
import jax.numpy as jnp
from jax import lax
from jax.experimental import pallas as pl
from jax.experimental.pallas import tpu as pltpu
from jax.experimental.pallas import tpu_sc as plsc

N_NODES = 50000
N_EDGES = 800000
D = 64
H = 32

NS = 16
CHUNK = 128
PHASES = 12
EDGES_TILE = N_EDGES // NS
NG = EDGES_TILE // CHUNK
NG_PIPE = (NG // PHASES) * PHASES
TAIL = EDGES_TILE - NG * CHUNK
E4_HALF = N_EDGES // 4

ROWS_PER_TILE = (N_NODES // (8 * NS)) * 8
ROWS_REM = N_NODES - NS * ROWS_PER_TILE
ZR = 240
ACC_ROWS = N_NODES


def _node_proj(x, wp, bp, wm):
    n, d_in = x.shape
    bn = 5000
    nb = n // bn

    def body(x_ref, wp_ref, bp_ref, wm_ref, hv_ref, pcat_ref):
        i = pl.program_id(0)
        hv = jnp.maximum(
            jnp.dot(x_ref[...], wp_ref[...], preferred_element_type=jnp.float32)
            + bp_ref[...], 0.0)
        hv_ref[...] = hv
        p = jnp.dot(hv, wm_ref[...], preferred_element_type=jnp.float32)

        @pl.when(i < nb)
        def _():
            pcat_ref[...] = p[:, :H]

        @pl.when(i >= nb)
        def _():
            pcat_ref[...] = p[:, H:]

    return pl.pallas_call(
        body,
        grid=(2 * nb,),
        in_specs=[
            pl.BlockSpec((bn, d_in), lambda i: (i % nb, 0)),
            pl.BlockSpec((d_in, D), lambda i: (0, 0)),
            pl.BlockSpec((1, D), lambda i: (0, 0)),
            pl.BlockSpec((D, D), lambda i: (0, 0)),
        ],
        out_specs=[
            pl.BlockSpec((bn, D), lambda i: (i % nb, 0)),
            pl.BlockSpec((bn, H), lambda i: (i, 0)),
        ],
        out_shape=[
            jax.ShapeDtypeStruct((n, D), jnp.float32),
            jax.ShapeDtypeStruct((2 * n, H), jnp.float32),
        ],
    )(x, wp, bp, wm)


def _edge_tbl(ef4, wlo, whi, blo, bhi):
    bn = 2000
    nb = E4_HALF // bn

    def body(ef_ref, wlo_ref, whi_ref, blo_ref, bhi_ref, out_ref):
        j = pl.program_id(0)
        x = ef_ref[...]

        @pl.when(j < nb)
        def _():
            out_ref[...] = jnp.dot(
                x, wlo_ref[...], preferred_element_type=jnp.float32
            ) + blo_ref[...]

        @pl.when(j >= nb)
        def _():
            out_ref[...] = jnp.dot(
                x, whi_ref[...], preferred_element_type=jnp.float32
            ) + bhi_ref[...]

    return pl.pallas_call(
        body,
        grid=(2 * nb,),
        in_specs=[
            pl.BlockSpec((bn, 48), lambda j: (j % nb, 0)),
            pl.BlockSpec((48, 128), lambda j: (0, 0)),
            pl.BlockSpec((48, 128), lambda j: (0, 0)),
            pl.BlockSpec((1, 128), lambda j: (0, 0)),
            pl.BlockSpec((1, 128), lambda j: (0, 0)),
        ],
        out_specs=pl.BlockSpec((bn, 128), lambda j: (j, 0)),
        out_shape=jax.ShapeDtypeStruct((2 * E4_HALF, 128), jnp.float32),
    )(ef4, wlo, whi, blo, bhi)


def _update(hv, slo, shi, w1, w2a, w2b, b, wp):
    n = hv.shape[0]
    bn = 5000
    nb = n // bn

    def body(hv_ref, slo_ref, shi_ref, w1_ref, w2a_ref, w2b_ref, b_ref, wp_ref,
             o_ref, pcat_ref):
        i = pl.program_id(0)
        acc = jnp.dot(hv_ref[...], w1_ref[...], preferred_element_type=jnp.float32)
        acc += jnp.dot(slo_ref[...], w2a_ref[...], preferred_element_type=jnp.float32)
        acc += jnp.dot(shi_ref[...], w2b_ref[...], preferred_element_type=jnp.float32)
        hv_new = jnp.maximum(acc + b_ref[...], 0.0)
        o_ref[...] = hv_new
        p = jnp.dot(hv_new, wp_ref[...], preferred_element_type=jnp.float32)

        @pl.when(i < nb)
        def _():
            pcat_ref[...] = p[:, :H]

        @pl.when(i >= nb)
        def _():
            pcat_ref[...] = p[:, H:]

    return pl.pallas_call(
        body,
        grid=(2 * nb,),
        in_specs=[
            pl.BlockSpec((bn, D), lambda i: (i % nb, 0)),
            pl.BlockSpec((bn, H), lambda i: (i % nb, 0)),
            pl.BlockSpec((bn, H), lambda i: (i % nb, 0)),
            pl.BlockSpec((D, D), lambda i: (0, 0)),
            pl.BlockSpec((H, D), lambda i: (0, 0)),
            pl.BlockSpec((H, D), lambda i: (0, 0)),
            pl.BlockSpec((1, D), lambda i: (0, 0)),
            pl.BlockSpec((D, D), lambda i: (0, 0)),
        ],
        out_specs=[
            pl.BlockSpec((bn, D), lambda i: (i % nb, 0)),
            pl.BlockSpec((bn, H), lambda i: (i, 0)),
        ],
        out_shape=[
            jax.ShapeDtypeStruct((n, D), jnp.float32),
            jax.ShapeDtypeStruct((2 * n, H), jnp.float32),
        ],
    )(hv, slo, shi, w1, w2a, w2b, b, wp)


def _final(hv, hlo, hhi, ws):
    n = hv.shape[0]
    bn = 5000

    def body(hv_ref, hlo_ref, hhi_ref, ws_ref, o_ref):
        s = jnp.dot(hv_ref[...], ws_ref[...], preferred_element_type=jnp.float32)
        o_ref[...] = jnp.concatenate(
            [hlo_ref[...] * s[:, :H], hhi_ref[...] * s[:, H:]], axis=1)

    return pl.pallas_call(
        body,
        grid=(n // bn,),
        in_specs=[
            pl.BlockSpec((bn, D), lambda i: (i, 0)),
            pl.BlockSpec((bn, H), lambda i: (i, 0)),
            pl.BlockSpec((bn, H), lambda i: (i, 0)),
            pl.BlockSpec((D, D), lambda i: (0, 0)),
        ],
        out_specs=pl.BlockSpec((bn, D), lambda i: (i, 0)),
        out_shape=jax.ShapeDtypeStruct((n, D), jnp.float32),
    )(hv, hlo, hhi, ws)


def _make_sc_segment(mul: bool):
    mesh = plsc.VectorSubcoreMesh(core_axis_name="c", subcore_axis_name="s",
                                  num_cores=2, num_subcores=NS)

    @functools.partial(
        pl.kernel,
        out_type=[
            jax.ShapeDtypeStruct((N_NODES, H), jnp.float32),
            jax.ShapeDtypeStruct((N_NODES, H), jnp.float32),
        ],
        mesh=mesh,
        compiler_params=pltpu.CompilerParams(use_tc_tiling_on_sc=False),
        scratch_types=[
            pltpu.VMEM((2, CHUNK), jnp.int32),
            pltpu.VMEM((4, CHUNK), jnp.int32),
            pltpu.VMEM((TAIL,), jnp.int32),
            pltpu.VMEM((TAIL,), jnp.int32),
            pltpu.VMEM((3, CHUNK, H), jnp.float32),
            pltpu.VMEM((2, CHUNK // 4, 128), jnp.float32),
            pltpu.VMEM((ZR, H), jnp.float32),
            pltpu.VMEM_SHARED((ACC_ROWS, H), jnp.float32),
            [pltpu.SemaphoreType.DMA] * 3,
            [pltpu.SemaphoreType.DMA] * 2,
            [pltpu.SemaphoreType.DMA] * 3,
            [pltpu.SemaphoreType.DMA] * 2,
            [pltpu.SemaphoreType.DMA] * 4,
        ],
    )
    def seg(src_ref, dst_ref, tcat, ecat, out_lo, out_hi,
            sidx, didx, tsidx, tdidx, prow, erow, zrow, acc,
            gsems, esems, ssems, sisems, disems):
        c = lax.axis_index("c")
        s = lax.axis_index("s")

        zeros16 = jnp.zeros((16,), jnp.float32)

        @pl.loop(0, ZR)
        def _zfill(i):
            zrow[i, pl.ds(0, 16)] = zeros16
            zrow[i, pl.ds(16, 16)] = zeros16

        def zcopy(dst):
            return pltpu.make_async_copy(zrow, dst, gsems[0])

        for j in range(ROWS_PER_TILE // ZR):
            zcopy(acc.at[pl.ds(s * ROWS_PER_TILE + j * ZR, ZR)]).start()

        @pl.when(s == 0)
        def _ztail():
            pltpu.async_copy(zrow.at[pl.ds(0, ROWS_REM)],
                             acc.at[pl.ds(NS * ROWS_PER_TILE, ROWS_REM)],
                             gsems[0])

        for j in range(ROWS_PER_TILE // ZR):
            zcopy(acc.at[pl.ds(s * ROWS_PER_TILE + j * ZR, ZR)]).wait()

        @pl.when(s == 0)
        def _ztail2():
            pltpu.make_async_copy(
                zrow.at[pl.ds(0, ROWS_REM)],
                acc.at[pl.ds(NS * ROWS_PER_TILE, ROWS_REM)], gsems[0]).wait()

        off16 = jnp.full((16,), c * N_NODES, dtype=jnp.int32)
        ebase = s * EDGES_TILE
        ebase4 = c * E4_HALF + s * (EDGES_TILE // 4)

        def isrc(g, slot):
            return pltpu.make_async_copy(
                src_ref.at[pl.ds(ebase + g * CHUNK, CHUNK)], sidx.at[slot],
                sisems[slot])

        def idst(g, slot):
            return pltpu.make_async_copy(
                dst_ref.at[pl.ds(ebase + g * CHUNK, CHUNK)], didx.at[slot],
                disems[slot])

        def shift_src(slot):
            for j in range(CHUNK // 16):
                sidx[slot, pl.ds(j * 16, 16)] = (
                    sidx[slot, pl.ds(j * 16, 16)] + off16)

        def gather(sp, si):
            return pltpu.make_async_copy(
                tcat.at[sidx.at[si]], prow.at[sp], gsems[sp])

        def erows(g, slot):
            return pltpu.make_async_copy(
                ecat.at[pl.ds(ebase4 + g * (CHUNK // 4), CHUNK // 4)],
                erow.at[slot], esems[slot])

        def scat_start(sp, slot4):
            pltpu.async_copy(prow.at[sp], acc.at[didx.at[slot4]],
                             ssems[sp], add=True)

        def scat_wait(sp, slot4):
            pltpu.make_async_copy(
                prow.at[sp], acc.at[didx.at[slot4]], ssems[sp]).wait()

        def combine(sp, se):
            @pl.loop(0, CHUNK // 4, unroll=1)
            def _rows(r):
                for jj in range(4):
                    for off in (0, 16):
                        e = erow[se, r, pl.ds(32 * jj + off, 16)]
                        pv = prow[sp, 4 * r + jj, pl.ds(off, 16)]
                        if mul:
                            prow[sp, 4 * r + jj, pl.ds(off, 16)] = e * pv
                        else:
                            prow[sp, 4 * r + jj, pl.ds(off, 16)] = (
                                jnp.maximum(e + pv, 0.0))

        plsc.subcore_barrier()

        isrc(0, 0).start()
        idst(0, 0).start()
        isrc(1, 1).start()
        idst(1, 1).start()
        isrc(0, 0).wait()
        shift_src(0)
        gather(0, 0).start()
        erows(0, 0).start()

        @pl.loop(0, NG_PIPE // PHASES)
        def _groups(k):
            for p in range(PHASES):
                s2 = p % 2
                s3 = p % 3
                s4 = p % 4
                g = k * PHASES + p

                gather(s3, s2).wait()
                erows(g, s2).wait()

                @pl.when(g >= 2)
                def _():
                    scat_wait((p + 1) % 3, (p + 2) % 4)

                @pl.when(g + 2 < NG_PIPE)
                def _():
                    isrc(g + 2, s2).start()
                    idst(g + 2, (p + 2) % 4).start()

                @pl.when(g + 1 < NG_PIPE)
                def _():
                    isrc(g + 1, (p + 1) % 2).wait()
                    shift_src((p + 1) % 2)
                    gather((p + 1) % 3, (p + 1) % 2).start()
                    erows(g + 1, (p + 1) % 2).start()

                combine(s3, s2)
                idst(g, s4).wait()
                scat_start(s3, s4)

        scat_wait((NG_PIPE - 2) % 3, (NG_PIPE - 2) % 4)
        scat_wait((NG_PIPE - 1) % 3, (NG_PIPE - 1) % 4)

        @pl.loop(NG_PIPE, NG)
        def _tailg(g):
            pltpu.sync_copy(src_ref.at[pl.ds(ebase + g * CHUNK, CHUNK)],
                            sidx.at[0])
            pltpu.sync_copy(dst_ref.at[pl.ds(ebase + g * CHUNK, CHUNK)],
                            didx.at[0])
            shift_src(0)
            gather(0, 0).start()
            erows(g, 0).start()
            gather(0, 0).wait()
            erows(g, 0).wait()
            combine(0, 0)
            scat_start(0, 0)
            scat_wait(0, 0)

        toff = ebase + NG * CHUNK
        pltpu.sync_copy(src_ref.at[pl.ds(toff, TAIL)], tsidx)
        pltpu.sync_copy(dst_ref.at[pl.ds(toff, TAIL)], tdidx)
        for j in range(TAIL // 16):
            tsidx[pl.ds(j * 16, 16)] = tsidx[pl.ds(j * 16, 16)] + off16
        tg = pltpu.make_async_copy(
            tcat.at[tsidx], prow.at[0, pl.ds(0, TAIL)], gsems[0])
        te = pltpu.make_async_copy(
            ecat.at[pl.ds(ebase4 + NG * (CHUNK // 4), TAIL // 4)],
            erow.at[0, pl.ds(0, TAIL // 4)], esems[0])
        tg.start()
        te.start()
        tg.wait()
        te.wait()

        @pl.loop(0, TAIL // 4)
        def _trows(r):
            for jj in range(4):
                for off in (0, 16):
                    e = erow[0, r, pl.ds(32 * jj + off, 16)]
                    pv = prow[0, 4 * r + jj, pl.ds(off, 16)]
                    if mul:
                        prow[0, 4 * r + jj, pl.ds(off, 16)] = e * pv
                    else:
                        prow[0, 4 * r + jj, pl.ds(off, 16)] = (
                            jnp.maximum(e + pv, 0.0))

        ts = pltpu.make_async_copy(
            prow.at[0, pl.ds(0, TAIL)], acc.at[tdidx], ssems[0])
        pltpu.async_copy(prow.at[0, pl.ds(0, TAIL)], acc.at[tdidx],
                         ssems[0], add=True)
        ts.wait()

        plsc.subcore_barrier()
        r0 = s * ROWS_PER_TILE
        rtail = NS * ROWS_PER_TILE

        @pl.when(c == 0)
        def _():
            pltpu.sync_copy(acc.at[pl.ds(r0, ROWS_PER_TILE)],
                            out_lo.at[pl.ds(r0, ROWS_PER_TILE)])

            @pl.when(s == 0)
            def _():
                pltpu.sync_copy(acc.at[pl.ds(rtail, ROWS_REM)],
                                out_lo.at[pl.ds(rtail, ROWS_REM)])

        @pl.when(c == 1)
        def _():
            pltpu.sync_copy(acc.at[pl.ds(r0, ROWS_PER_TILE)],
                            out_hi.at[pl.ds(r0, ROWS_PER_TILE)])

            @pl.when(s == 0)
            def _():
                pltpu.sync_copy(acc.at[pl.ds(rtail, ROWS_REM)],
                                out_hi.at[pl.ds(rtail, ROWS_REM)])

    return seg


_seg_sum_relu = _make_sc_segment(mul=False)
_seg_sum_mul = _make_sc_segment(mul=True)


def kernel(node_feats, edge_feats, edge_index, W_proj_in, b_proj_in,
           W_msg, b_msg, W_new, b_new, W_node_msg, W_edge_msg, W_self):
    src = edge_index[0].astype(jnp.int32)
    dst = edge_index[1].astype(jnp.int32)
    ef4 = edge_feats.reshape(N_EDGES // 4, 48)

    bd = jax.scipy.linalg.block_diag
    we, wm = W_msg[D:], W_edge_msg
    welo = bd(*([we[:, :H]] * 4))
    wehi = bd(*([we[:, H:]] * 4))
    wmlo = bd(*([wm[:, :H]] * 4))
    wmhi = bd(*([wm[:, H:]] * 4))
    belo = jnp.tile(b_msg[:H], 4).reshape(1, 128)
    behi = jnp.tile(b_msg[H:], 4).reshape(1, 128)

    hv, pcat = _node_proj(node_feats, W_proj_in, b_proj_in.reshape(1, D),
                          W_msg[:D])
    ecat = _edge_tbl(ef4, welo, wehi, belo, behi)
    zb = jnp.zeros((1, 128), jnp.float32)
    mcat = _edge_tbl(ef4, wmlo, wmhi, zb, zb)

    for layer in range(4):
        slo, shi = _seg_sum_relu(src, dst, pcat, ecat)
        wp = W_msg[:D] if layer < 3 else W_node_msg
        hv, pcat = _update(hv, slo, shi, W_new[:D], W_new[D:D + H],
                           W_new[D + H:], b_new.reshape(1, D), wp)

    hlo, hhi = _seg_sum_mul(src, dst, pcat, mcat)
    return _final(hv, hlo, hhi, W_self)
